# Initial kernel scaffold; baseline (speedup 1.0000x reference)
import jax
import jax.numpy as jnp
from jax.experimental import pallas as pl


def kernel(x_cat, tables, W1, b1, W2, b2):
    raise NotImplementedError("write your pallas kernel here")


# speedup vs baseline: 2.0161x; 2.0161x over previous
import functools
import math

import jax
import jax.numpy as jnp
from jax import lax
from jax.experimental import pallas as pl
from jax.experimental.pallas import tpu as pltpu
from jax.experimental.pallas import tpu_sc as plsc

F = 26
V = 100000
D = 32
B = 16384
P = 128

NC = 2
NS = 16
NW = NC * NS
ROWS = B * F
ROWS_PER_W = ROWS // NW
CHUNK = 128
CHUNKS_PER_W = ROWS_PER_W // CHUNK
WAVE = 8
WAVES = CHUNKS_PER_W // WAVE


def _sc_gather(idx2d_hbm, tab_hbm, out_hbm, idx_v, rows_v, gsem, osem):
    wid = lax.axis_index("s") * NC + lax.axis_index("c")
    chunk0 = wid * CHUNKS_PER_W
    pltpu.sync_copy(idx2d_hbm.at[pl.ds(chunk0, CHUNKS_PER_W)], idx_v)

    def wave_body(w, carry):
        cbase = w * WAVE
        gathers = []
        for j in range(WAVE):
            gathers.append(
                pltpu.async_copy(tab_hbm.at[idx_v.at[cbase + j]],
                                 rows_v.at[j], gsem))
        for g in gathers:
            g.wait()
        outs = []
        for j in range(WAVE):
            rowbase = (chunk0 + cbase + j) * CHUNK
            outs.append(
                pltpu.async_copy(rows_v.at[j],
                                 out_hbm.at[pl.ds(rowbase, CHUNK)], osem))
        for o in outs:
            o.wait()
        return carry

    lax.fori_loop(0, WAVES, wave_body, 0, unroll=False)


def _gather_concat(tab_flat, idx2d):
    mesh = plsc.VectorSubcoreMesh(core_axis_name="c", subcore_axis_name="s",
                                  num_cores=NC, num_subcores=NS)
    fn = pl.kernel(
        _sc_gather,
        out_type=jax.ShapeDtypeStruct((ROWS, D), jnp.float32),
        mesh=mesh,
        scratch_types=[
            pltpu.VMEM((CHUNKS_PER_W, CHUNK), jnp.int32),
            pltpu.VMEM((WAVE, CHUNK, D), jnp.float32),
            pltpu.SemaphoreType.DMA,
            pltpu.SemaphoreType.DMA,
        ],
        compiler_params=pltpu.CompilerParams(use_tc_tiling_on_sc=False),
    )
    return fn(idx2d, tab_flat)


_SQRT_HALF = math.sqrt(0.5)


def _mlp_block(x_ref, w1_ref, b1_ref, w2_ref, b2_ref, o_ref):
    h = jnp.dot(x_ref[...], w1_ref[...],
                preferred_element_type=jnp.float32) + b1_ref[...]
    h = 0.5 * h * (1.0 + lax.erf(h * _SQRT_HALF))
    o_ref[...] = jnp.dot(h, w2_ref[...],
                         preferred_element_type=jnp.float32) + b2_ref[...]


BM = 1024


def _mlp(x, W1, b1, W2, b2):
    CD = F * D
    grid = (B // BM,)
    return pl.pallas_call(
        _mlp_block,
        grid=grid,
        in_specs=[
            pl.BlockSpec((BM, CD), lambda i: (i, 0)),
            pl.BlockSpec((CD, P), lambda i: (0, 0)),
            pl.BlockSpec((1, P), lambda i: (0, 0)),
            pl.BlockSpec((P, P), lambda i: (0, 0)),
            pl.BlockSpec((1, P), lambda i: (0, 0)),
        ],
        out_specs=pl.BlockSpec((BM, P), lambda i: (i, 0)),
        out_shape=jax.ShapeDtypeStruct((B, P), jnp.float32),
    )(x, W1, b1.reshape(1, P), W2, b2.reshape(1, P))


def kernel(x_cat, tables, W1, b1, W2, b2):
    tab_flat = tables.reshape(F * V, D)
    offs = (jnp.arange(F, dtype=jnp.int32) * V)[None, :]
    idx2d = (x_cat.astype(jnp.int32) + offs).reshape(ROWS // CHUNK, CHUNK)
    emb = _gather_concat(tab_flat, idx2d)
    x = emb.reshape(B, F * D)
    return _mlp(x, W1, b1, W2, b2)


# speedup vs baseline: 12.1663x; 6.0346x over previous
import math

import jax
import jax.numpy as jnp
from jax import lax
from jax.experimental import pallas as pl
from jax.experimental.pallas import tpu as pltpu
from jax.experimental.pallas import tpu_sc as plsc

F = 26
V = 100000
D = 32
B = 16384
P = 128
CD = F * D
HCD = CD // 2

NC = 2
NS = 16
NW = NC * NS
UNITS = HCD // NW
LANES = 16
OC = 4096
OCHUNKS = B // OC


def _make_sc_gather(half):
    base = half * HCD

    def sc_gather(tab_hbm, idx_hbm, out_hbm,
                  row_v, idx_v, oc0, oc1, gsem, osem):
        wid = lax.axis_index("s") * NC + lax.axis_index("c")
        r0 = wid * UNITS
        ocs = (oc0, oc1)

        def drain(unit, c, oc_b):
            return pltpu.make_async_copy(
                oc_b, out_hbm.at[r0 + unit, pl.ds(c * OC, OC)], osem)

        def unit_body(unit, last_f):
            g = base + r0 + unit
            f = g // D

            def reload(_):
                pltpu.sync_copy(idx_hbm.at[f], idx_v)
                return f

            last_f = lax.cond(f != last_f, reload, lambda _: last_f, 0)
            pltpu.sync_copy(tab_hbm.at[f, g % D], row_v)

            for c in range(OCHUNKS):
                oc_b = ocs[c % 2]
                if c < 2:
                    def wait_prev(_):
                        drain(unit - 1, OCHUNKS - 2 + c, oc_b).wait()
                        return 0

                    lax.cond(unit > 0, wait_prev, lambda _: 0, 0)
                else:
                    drain(unit, c - 2, oc_b).wait()

                cbase = c * OC

                @plsc.parallel_loop(0, OC // LANES, unroll=16)
                def gather16(i):
                    s = pl.ds(cbase + i * LANES, LANES)
                    idx16 = idx_v[s]
                    oc_b[pl.ds(i * LANES, LANES)] = plsc.load_gather(
                        row_v, [idx16])

                drain(unit, c, oc_b).start()
            return last_f

        lax.fori_loop(0, UNITS, unit_body, jnp.int32(-1))
        drain(UNITS - 1, OCHUNKS - 2, ocs[0]).wait()
        drain(UNITS - 1, OCHUNKS - 1, ocs[1]).wait()

    return sc_gather


def _gather_half(tab_t, idx_t, half):
    mesh = plsc.VectorSubcoreMesh(core_axis_name="c", subcore_axis_name="s",
                                  num_cores=NC, num_subcores=NS)
    fn = pl.kernel(
        _make_sc_gather(half),
        out_type=jax.ShapeDtypeStruct((HCD, B), jnp.float32),
        mesh=mesh,
        scratch_types=[
            pltpu.VMEM((V,), jnp.float32),
            pltpu.VMEM((B,), jnp.int32),
            pltpu.VMEM((OC,), jnp.float32),
            pltpu.VMEM((OC,), jnp.float32),
            pltpu.SemaphoreType.DMA,
            pltpu.SemaphoreType.DMA,
        ],
        compiler_params=pltpu.CompilerParams(use_tc_tiling_on_sc=True,
                                             needs_layout_passes=False),
        name=f"sc_gather_half{half}",
    )
    return fn(tab_t, idx_t)


_SQRT_HALF = math.sqrt(0.5)
BM = 4096


def _mm1_block(xt_ref, w_ref, o_ref):
    o_ref[...] = lax.dot_general(xt_ref[...], w_ref[...],
                                 (((0,), (0,)), ((), ())),
                                 preferred_element_type=jnp.float32)


def _mm2_block(xt_ref, part_ref, w1_ref, b1_ref, w2_ref, b2_ref, o_ref):
    h = part_ref[...] + lax.dot_general(
        xt_ref[...], w1_ref[...], (((0,), (0,)), ((), ())),
        preferred_element_type=jnp.float32) + b1_ref[...]
    h = 0.5 * h * (1.0 + lax.erf(h * _SQRT_HALF))
    o_ref[...] = jnp.dot(h, w2_ref[...],
                         preferred_element_type=jnp.float32) + b2_ref[...]


def _mlp_stage1(xt1, W1a):
    return pl.pallas_call(
        _mm1_block,
        grid=(B // BM,),
        in_specs=[
            pl.BlockSpec((HCD, BM), lambda i: (0, i)),
            pl.BlockSpec((HCD, P), lambda i: (0, 0)),
        ],
        out_specs=pl.BlockSpec((BM, P), lambda i: (i, 0)),
        out_shape=jax.ShapeDtypeStruct((B, P), jnp.float32),
    )(xt1, W1a)


def _mlp_stage2(xt2, part, W1b, b1, W2, b2):
    return pl.pallas_call(
        _mm2_block,
        grid=(B // BM,),
        in_specs=[
            pl.BlockSpec((HCD, BM), lambda i: (0, i)),
            pl.BlockSpec((BM, P), lambda i: (i, 0)),
            pl.BlockSpec((HCD, P), lambda i: (0, 0)),
            pl.BlockSpec((1, P), lambda i: (0, 0)),
            pl.BlockSpec((P, P), lambda i: (0, 0)),
            pl.BlockSpec((1, P), lambda i: (0, 0)),
        ],
        out_specs=pl.BlockSpec((BM, P), lambda i: (i, 0)),
        out_shape=jax.ShapeDtypeStruct((B, P), jnp.float32),
    )(xt2, part, W1b, b1.reshape(1, P), W2, b2.reshape(1, P))


def kernel(x_cat, tables, W1, b1, W2, b2):
    tab_t = tables.transpose(0, 2, 1)
    idx_t = x_cat.astype(jnp.int32).T
    xt1 = _gather_half(tab_t, idx_t, 0)
    part = _mlp_stage1(xt1, W1[:HCD])
    xt2 = _gather_half(tab_t, idx_t, 1)
    return _mlp_stage2(xt2, part, W1[HCD:], b1, W2, b2)


# speedup vs baseline: 12.7287x; 1.0462x over previous
import math

import jax
import jax.numpy as jnp
from jax import lax
from jax.experimental import pallas as pl
from jax.experimental.pallas import tpu as pltpu
from jax.experimental.pallas import tpu_sc as plsc

F = 26
V = 100000
D = 32
B = 16384
P = 128
CD = F * D

NC = 2
NS = 16
NW = NC * NS
UNITS_PER_W = CD // NW
PAIRS = UNITS_PER_W // 2
LANES = 16
OC = 4096
OCHUNKS = B // OC


def _sc_gather_t(tab_hbm, idx_hbm, out_hbm,
                 row_v, idx_v, oc0, oc1, gsem, osem):
    wid = lax.axis_index("s") * NC + lax.axis_index("c")
    g0 = wid * UNITS_PER_W
    ocs = (oc0, oc1)

    def drain(unit, c, oc_b):
        return pltpu.make_async_copy(
            oc_b, out_hbm.at[g0 + unit, pl.ds(c * OC, OC)], osem)

    def pair_body(p, last_f):
        for ul in range(2):
            unit = p * 2 + ul
            g = g0 + unit
            f = g // D

            def reload(_):
                pltpu.sync_copy(idx_hbm.at[f], idx_v)
                return f

            last_f = lax.cond(f != last_f, reload, lambda _: last_f, 0)
            pltpu.sync_copy(tab_hbm.at[f, g % D], row_v)

            for c in range(OCHUNKS):
                oc_b = ocs[c % 2]
                if ul == 0 and c < 2:
                    def wait_prev(_):
                        drain(unit - 1, OCHUNKS - 2 + c, oc_b).wait()
                        return 0
                    lax.cond(p > 0, wait_prev, lambda _: 0, 0)
                else:
                    drain(unit, c - 2, oc_b).wait() if c >= 2 else \
                        drain(unit - 1, OCHUNKS - 2 + c, oc_b).wait()

                cbase = c * OC

                @plsc.parallel_loop(0, OC // LANES, unroll=16)
                def gather16(i):
                    s = pl.ds(cbase + i * LANES, LANES)
                    idx16 = idx_v[s]
                    oc_b[pl.ds(i * LANES, LANES)] = plsc.load_gather(
                        row_v, [idx16])

                drain(unit, c, oc_b).start()
        return last_f

    lax.fori_loop(0, PAIRS, pair_body, jnp.int32(-1))
    drain(UNITS_PER_W - 1, OCHUNKS - 2, ocs[0]).wait()
    drain(UNITS_PER_W - 1, OCHUNKS - 1, ocs[1]).wait()


def _gather_concat_t(tab_t, idx_t):
    mesh = plsc.VectorSubcoreMesh(core_axis_name="c", subcore_axis_name="s",
                                  num_cores=NC, num_subcores=NS)
    fn = pl.kernel(
        _sc_gather_t,
        out_type=jax.ShapeDtypeStruct((CD, B), jnp.float32),
        mesh=mesh,
        scratch_types=[
            pltpu.VMEM((V,), jnp.float32),
            pltpu.VMEM((B,), jnp.int32),
            pltpu.VMEM((OC,), jnp.float32),
            pltpu.VMEM((OC,), jnp.float32),
            pltpu.SemaphoreType.DMA,
            pltpu.SemaphoreType.DMA,
        ],
        compiler_params=pltpu.CompilerParams(use_tc_tiling_on_sc=True,
                                             needs_layout_passes=False),
    )
    return fn(tab_t, idx_t)


_SQRT_HALF = math.sqrt(0.5)


def _mlp_block(xt_ref, w1_ref, b1_ref, w2_ref, b2_ref, o_ref):
    h = lax.dot_general(xt_ref[...], w1_ref[...],
                        (((0,), (0,)), ((), ())),
                        preferred_element_type=jnp.float32) + b1_ref[...]
    h = 0.5 * h * (1.0 + lax.erf(h * _SQRT_HALF))
    o_ref[...] = jnp.dot(h, w2_ref[...],
                         preferred_element_type=jnp.float32) + b2_ref[...]


BM = 4096


def _mlp_t(xt, W1, b1, W2, b2):
    return pl.pallas_call(
        _mlp_block,
        grid=(B // BM,),
        in_specs=[
            pl.BlockSpec((CD, BM), lambda i: (0, i)),
            pl.BlockSpec((CD, P), lambda i: (0, 0)),
            pl.BlockSpec((1, P), lambda i: (0, 0)),
            pl.BlockSpec((P, P), lambda i: (0, 0)),
            pl.BlockSpec((1, P), lambda i: (0, 0)),
        ],
        out_specs=pl.BlockSpec((BM, P), lambda i: (i, 0)),
        out_shape=jax.ShapeDtypeStruct((B, P), jnp.float32),
    )(xt, W1, b1.reshape(1, P), W2, b2.reshape(1, P))


def kernel(x_cat, tables, W1, b1, W2, b2):
    tab_t = tables.transpose(0, 2, 1)
    idx_t = x_cat.astype(jnp.int32).T
    xt = _gather_concat_t(tab_t, idx_t)
    return _mlp_t(xt, W1, b1, W2, b2)
